# baseline (speedup 1.0000x reference)
import jax
import jax.numpy as jnp
from jax import lax
from jax.experimental import pallas as pl
from jax.experimental.pallas import tpu as pltpu

B, Q, L = 64, 100, 256
BB = 32


def _postproc_kernel(logits_ref, boxes_ref, ts_ref, posmap_ref, out_ref):
    x = logits_ref[...]
    m = jnp.max(x, axis=-1, keepdims=True)
    s = jnp.sum(jnp.exp(x - m), axis=-1, keepdims=True)
    pos = posmap_ref[...][:, None, :] > 1e-6
    mm = jnp.max(jnp.where(pos, x, -jnp.inf), axis=-1, keepdims=True)
    score = jnp.exp(mm - m) / s

    ts = ts_ref[...].astype(jnp.float32)
    img_h = ts[:, 0:1][:, None, :]
    img_w = ts[:, 1:2][:, None, :]

    bx = boxes_ref[...]
    lane = lax.broadcasted_iota(jnp.int32, (1, 1, 4), 2)
    axscale = jnp.where(lane % 2 == 0, img_w, img_h)
    kk = lax.broadcasted_iota(jnp.int32, (4, 4), 0)
    cc = lax.broadcasted_iota(jnp.int32, (4, 4), 1)
    cvt = jnp.where(
        (cc % 2) == (kk % 2),
        jnp.where(kk < 2, 1.0, jnp.where(cc < 2, -0.5, 0.5)),
        0.0,
    )

    score_row = jnp.swapaxes(score, 1, 2)
    ii = lax.broadcasted_iota(jnp.int32, (1, Q, Q), 1)
    jj = lax.broadcasted_iota(jnp.int32, (1, Q, Q), 2)

    beats = (score_row > score) | ((score_row == score) & (jj < ii))
    rank_row = (Q - 1) - jnp.sum(
        beats.astype(jnp.int32), axis=1, keepdims=True
    )

    rr = lax.broadcasted_iota(jnp.int32, (1, Q, 1), 1)
    take = (rank_row == rr).astype(jnp.float32)

    g = lax.dot_general(
        take,
        bx,
        dimension_numbers=(((2,), (1,)), ((0,), (0,))),
        preferred_element_type=jnp.float32,
    )
    xyxy = lax.dot_general(
        g,
        cvt,
        dimension_numbers=(((2,), (0,)), ((), ())),
        preferred_element_type=jnp.float32,
    )
    out_ref[...] = xyxy * axscale


def kernel(pred_logits, pred_boxes, target_sizes, positive_map, items_per_batch_element):
    del items_per_batch_element
    return pl.pallas_call(
        _postproc_kernel,
        grid=(B // BB,),
        in_specs=[
            pl.BlockSpec((BB, Q, L), lambda b: (b, 0, 0)),
            pl.BlockSpec((BB, Q, 4), lambda b: (b, 0, 0)),
            pl.BlockSpec((BB, 2), lambda b: (b, 0)),
            pl.BlockSpec((BB, L), lambda b: (b, 0)),
        ],
        out_specs=pl.BlockSpec((BB, Q, 4), lambda b: (b, 0, 0)),
        out_shape=jax.ShapeDtypeStruct((B, Q, 4), jnp.float32),
        compiler_params=pltpu.CompilerParams(
            dimension_semantics=("parallel",),
        ),
    )(pred_logits, pred_boxes, target_sizes, positive_map)
